# baseline (speedup 1.0000x reference)
import functools

import jax
import jax.numpy as jnp
from jax import lax
from jax.experimental import pallas as pl
from jax.experimental.pallas import tpu as pltpu
from jax.experimental.pallas import tpu_sc as plsc

_BB = 1024
_CP = 1008


def _main_body(x_ref, cm_ref, cnt_ref, o_ref, acc_ref):
    i = pl.program_id(0)
    x = x_ref[...]
    rowmax = jnp.max(x, axis=1, keepdims=True)
    cm = jnp.max(x - rowmax, axis=0, keepdims=True)

    @pl.when(i == 0)
    def _():
        acc_ref[...] = cm

    @pl.when(i != 0)
    def _():
        acc_ref[...] = jnp.maximum(acc_ref[...], cm)

    @pl.when(i == pl.num_programs(0) - 1)
    def _():
        c = cm_ref.shape[0]
        cnt = jnp.sum(cnt_ref[...], axis=0, keepdims=True)[:, :c]
        w = lax.dot_general(
            cnt,
            cm_ref[...],
            (((1,), (0,)), ((), ())),
            precision=lax.Precision.HIGHEST,
            preferred_element_type=jnp.float32,
        )
        mask = (acc_ref[...] == 0.0).astype(jnp.float32)
        o_ref[...] = -jnp.sum(w * mask, keepdims=True)


@functools.lru_cache(maxsize=None)
def _make_hist(nw, nc, nb):
    mesh = plsc.VectorSubcoreMesh(core_axis_name="c", subcore_axis_name="s")

    @functools.partial(
        pl.kernel,
        mesh=mesh,
        out_type=jax.ShapeDtypeStruct((nw, _CP), jnp.float32),
        scratch_types=[
            pltpu.VMEM((nb,), jnp.int32),
            pltpu.VMEM((_CP,), jnp.float32),
        ],
        compiler_params=pltpu.CompilerParams(needs_layout_passes=False),
    )
    def hist_k(labels_hbm, out_hbm, idx_v, hist_v):
        wid = lax.axis_index("s") * nc + lax.axis_index("c")
        pltpu.sync_copy(labels_hbm.at[pl.ds(wid * nb, nb)], idx_v)
        zeros = jnp.zeros((16,), jnp.float32)
        for j in range(_CP // 16):
            hist_v[pl.ds(j * 16, 16)] = zeros
        ones = jnp.ones((16,), jnp.float32)
        for i in range(nb // 16):
            plsc.addupdate_scatter(hist_v, [idx_v[pl.ds(i * 16, 16)]], ones)
        pltpu.sync_copy(hist_v, out_hbm.at[wid])

    return hist_k


def kernel(outputs, labels, cost_matrix):
    b, c = outputs.shape

    info = plsc.get_sparse_core_info()
    nw = info.num_cores * info.num_subcores
    cnt32 = _make_hist(nw, info.num_cores, b // nw)(labels)

    loss = pl.pallas_call(
        _main_body,
        grid=(b // _BB,),
        in_specs=[
            pl.BlockSpec((_BB, c), lambda i: (i, 0)),
            pl.BlockSpec((c, c), lambda i: (0, 0)),
            pl.BlockSpec(cnt32.shape, lambda i: (0, 0)),
        ],
        out_specs=pl.BlockSpec((1, 1), lambda i: (0, 0)),
        out_shape=jax.ShapeDtypeStruct((1, 1), jnp.float32),
        scratch_shapes=[pltpu.VMEM((1, c), jnp.float32)],
    )(outputs, cost_matrix, cnt32)

    return loss[0, 0]


# speedup vs baseline: 1.0244x; 1.0244x over previous
import functools

import jax
import jax.numpy as jnp
from jax import lax
from jax.experimental import pallas as pl
from jax.experimental.pallas import tpu as pltpu
from jax.experimental.pallas import tpu_sc as plsc

_BB = 2048
_CP = 1008


def _main_body(x_ref, cm_ref, cnt_ref, o_ref, acc_ref):
    i = pl.program_id(0)
    x = x_ref[...]
    rowmax = jnp.max(x, axis=1, keepdims=True)
    cm = jnp.max(x - rowmax, axis=0, keepdims=True)

    @pl.when(i == 0)
    def _():
        acc_ref[...] = cm

    @pl.when(i != 0)
    def _():
        acc_ref[...] = jnp.maximum(acc_ref[...], cm)

    @pl.when(i == pl.num_programs(0) - 1)
    def _():
        c = cm_ref.shape[0]
        cnt = jnp.sum(cnt_ref[...], axis=0, keepdims=True)[:, :c]
        w = lax.dot_general(
            cnt,
            cm_ref[...],
            (((1,), (0,)), ((), ())),
            precision=lax.Precision.HIGHEST,
            preferred_element_type=jnp.float32,
        )
        mask = (acc_ref[...] == 0.0).astype(jnp.float32)
        o_ref[...] = -jnp.sum(w * mask, keepdims=True)


@functools.lru_cache(maxsize=None)
def _make_hist(nw, nc, nb):
    mesh = plsc.VectorSubcoreMesh(core_axis_name="c", subcore_axis_name="s")

    @functools.partial(
        pl.kernel,
        mesh=mesh,
        out_type=jax.ShapeDtypeStruct((nw, _CP), jnp.float32),
        scratch_types=[
            pltpu.VMEM((nb,), jnp.int32),
            pltpu.VMEM((_CP,), jnp.float32),
        ],
        compiler_params=pltpu.CompilerParams(needs_layout_passes=False),
    )
    def hist_k(labels_hbm, out_hbm, idx_v, hist_v):
        wid = lax.axis_index("s") * nc + lax.axis_index("c")
        pltpu.sync_copy(labels_hbm.at[pl.ds(wid * nb, nb)], idx_v)
        zeros = jnp.zeros((16,), jnp.float32)
        for j in range(_CP // 16):
            hist_v[pl.ds(j * 16, 16)] = zeros
        ones = jnp.ones((16,), jnp.float32)
        for i in range(nb // 16):
            plsc.addupdate_scatter(hist_v, [idx_v[pl.ds(i * 16, 16)]], ones)
        pltpu.sync_copy(hist_v, out_hbm.at[wid])

    return hist_k


def kernel(outputs, labels, cost_matrix):
    b, c = outputs.shape

    info = plsc.get_sparse_core_info()
    nw = info.num_cores * info.num_subcores
    cnt32 = _make_hist(nw, info.num_cores, b // nw)(labels)

    loss = pl.pallas_call(
        _main_body,
        grid=(b // _BB,),
        in_specs=[
            pl.BlockSpec((_BB, c), lambda i: (i, 0)),
            pl.BlockSpec((c, c), lambda i: (0, 0)),
            pl.BlockSpec(cnt32.shape, lambda i: (0, 0)),
        ],
        out_specs=pl.BlockSpec((1, 1), lambda i: (0, 0)),
        out_shape=jax.ShapeDtypeStruct((1, 1), jnp.float32),
        scratch_shapes=[pltpu.VMEM((1, c), jnp.float32)],
    )(outputs, cost_matrix, cnt32)

    return loss[0, 0]
